# baseline (speedup 1.0000x reference)
import functools

import jax
import jax.numpy as jnp
from jax import lax
from jax.experimental import pallas as pl
from jax.experimental.pallas import tpu as pltpu
from jax.experimental.pallas import tpu_sc as plsc

B = 4096
K = 32
H = B // 2

_info = plsc.get_sparse_core_info()
_NC, _NS = _info.num_cores, _info.num_subcores
_NW = _NC * _NS
_BPW = B // _NW
_HPW = H // _NW
_SG = 4


def _splat(x, n=16):
    return jnp.full((n,), x, jnp.int32)


def _idx_parts(idx_ref, count):
    parts = []
    for h in range(count // 16):
        v = idx_ref[pl.ds(h * 16, 16)]
        parts.append((lax.shift_right_logical(v, 7),
                      lax.bitwise_and(v, _splat(127))))
    return parts


def _fire(tab, parts, g, q, buf, sem):
    tu = parts[(g * _SG) // 16][0]
    for k in range(_SG):
        lane = (g * _SG) % 16 + k
        off = pl.multiple_of(tu[lane] * 128, 128)
        pltpu.make_async_copy(
            tab.at[:, pl.ds(off, 128)], buf.at[q, k], sem).start()


def _drain_extract(tab, parts, g, q, buf, cols, sem, c_lo, c_hi):
    ru = parts[(g * _SG) // 16][1]
    for k in range(_SG):
        pltpu.make_async_copy(
            tab.at[:, pl.ds(0, 128)], buf.at[q, k], sem).wait()
    qv = _splat(q)
    for k in range(_SG):
        lane = (g * _SG) % 16 + k
        slot = _splat(k)
        jv = _splat(g * _SG + k)
        rm = _splat(ru[lane])
        lo = plsc.load_gather(buf, [qv, slot, c_lo, rm])
        hi = plsc.load_gather(buf, [qv, slot, c_hi, rm])
        plsc.store_scatter(cols, [c_lo, jv], lo)
        plsc.store_scatter(cols, [c_hi, jv], hi)


def _sc_a_body(user_idx, item_idx, utab_t, itab_t, out_ut, out_it,
               idx_u, idx_i, buf_u, buf_i, cols_u, cols_i, sem_u, sem_i):
    wid = lax.axis_index("s") * _NC + lax.axis_index("c")
    base_u = wid * _HPW
    base_i = wid * _BPW
    pltpu.sync_copy(user_idx.at[pl.ds(base_u, _HPW)], idx_u)
    pltpu.sync_copy(item_idx.at[pl.ds(base_i, _BPW)], idx_i)
    c_lo = lax.iota(jnp.int32, 16)
    c_hi = c_lo + 16
    parts_u = _idx_parts(idx_u, _HPW)
    parts_i = _idx_parts(idx_i, _BPW)
    nsg_u = _HPW // _SG
    nsg_i = _BPW // _SG

    _fire(itab_t, parts_i, 0, 0, buf_i, sem_i)
    _fire(utab_t, parts_u, 0, 0, buf_u, sem_u)
    for g in range(nsg_i):
        if g + 1 < nsg_i:
            _fire(itab_t, parts_i, g + 1, (g + 1) % 2, buf_i, sem_i)
        if g + 1 < nsg_u:
            _fire(utab_t, parts_u, g + 1, (g + 1) % 2, buf_u, sem_u)
        _drain_extract(itab_t, parts_i, g, g % 2, buf_i, cols_i, sem_i,
                       c_lo, c_hi)
        if g < nsg_u:
            _drain_extract(utab_t, parts_u, g, g % 2, buf_u, cols_u, sem_u,
                           c_lo, c_hi)

    pltpu.sync_copy(cols_u, out_ut.at[wid])
    pltpu.sync_copy(cols_i, out_it.at[:, pl.ds(base_i, _BPW)])


def _sc_b_body(user_idx, utab_t, out_ut, idx_u, buf_u, cols_u, sem_u):
    wid = lax.axis_index("s") * _NC + lax.axis_index("c")
    base_u = H + wid * _HPW
    pltpu.sync_copy(user_idx.at[pl.ds(base_u, _HPW)], idx_u)
    c_lo = lax.iota(jnp.int32, 16)
    c_hi = c_lo + 16
    parts_u = _idx_parts(idx_u, _HPW)
    nsg_u = _HPW // _SG

    _fire(utab_t, parts_u, 0, 0, buf_u, sem_u)
    for g in range(nsg_u):
        if g + 1 < nsg_u:
            _fire(utab_t, parts_u, g + 1, (g + 1) % 2, buf_u, sem_u)
        _drain_extract(utab_t, parts_u, g, g % 2, buf_u, cols_u, sem_u,
                       c_lo, c_hi)

    pltpu.sync_copy(cols_u, out_ut.at[wid])


_mesh = plsc.VectorSubcoreMesh(core_axis_name="c", subcore_axis_name="s")

_sc_a = functools.partial(
    pl.kernel,
    mesh=_mesh,
    out_type=(
        jax.ShapeDtypeStruct((_NW, K, _HPW), jnp.float32),
        jax.ShapeDtypeStruct((K, B), jnp.float32),
    ),
    scratch_types=[
        pltpu.VMEM((_HPW,), jnp.int32),
        pltpu.VMEM((_BPW,), jnp.int32),
        pltpu.VMEM((2, _SG, K, 128), jnp.float32),
        pltpu.VMEM((2, _SG, K, 128), jnp.float32),
        pltpu.VMEM((K, _HPW), jnp.float32),
        pltpu.VMEM((K, _BPW), jnp.float32),
        pltpu.SemaphoreType.DMA,
        pltpu.SemaphoreType.DMA,
    ],
    compiler_params=pltpu.CompilerParams(needs_layout_passes=False),
)(_sc_a_body)

_sc_b = functools.partial(
    pl.kernel,
    mesh=_mesh,
    out_type=jax.ShapeDtypeStruct((_NW, K, _HPW), jnp.float32),
    scratch_types=[
        pltpu.VMEM((_HPW,), jnp.int32),
        pltpu.VMEM((2, _SG, K, 128), jnp.float32),
        pltpu.VMEM((K, _HPW), jnp.float32),
        pltpu.SemaphoreType.DMA,
    ],
    compiler_params=pltpu.CompilerParams(needs_layout_passes=False),
)(_sc_b_body)


_BM = 512
_BN = 4096
_NMB = H // _BM


def _mm1_body(u_ref, i_ref, o_ref):
    o_ref[...] = lax.dot_general(
        u_ref[...], i_ref[...],
        dimension_numbers=(((0,), (0,)), ((), ())),
        preferred_element_type=jnp.float32,
    )


_mm1 = pl.pallas_call(
    _mm1_body,
    grid=(_NMB,),
    in_specs=[
        pl.BlockSpec((K, _BM), lambda i: (0, i)),
        pl.BlockSpec((K, _BN), lambda i: (0, 0)),
    ],
    out_specs=pl.BlockSpec((_BM, _BN), lambda i: (i, 0)),
    out_shape=jax.ShapeDtypeStruct((B, B), jnp.float32),
)


def _mm2_body(u_ref, i_ref, prev_ref, o_ref):
    del prev_ref
    o_ref[...] = lax.dot_general(
        u_ref[...], i_ref[...],
        dimension_numbers=(((0,), (0,)), ((), ())),
        preferred_element_type=jnp.float32,
    )


_mm2 = pl.pallas_call(
    _mm2_body,
    grid=(_NMB,),
    in_specs=[
        pl.BlockSpec((K, _BM), lambda i: (0, i)),
        pl.BlockSpec((K, _BN), lambda i: (0, 0)),
        pl.BlockSpec(memory_space=pltpu.MemorySpace.HBM),
    ],
    out_specs=pl.BlockSpec((_BM, _BN), lambda i: (i + _NMB, 0)),
    out_shape=jax.ShapeDtypeStruct((B, B), jnp.float32),
    input_output_aliases={2: 0},
)


def kernel(user, item, user_table, item_table):
    user = user.astype(jnp.int32)
    item = item.astype(jnp.int32)
    utab_t = user_table.T
    ut_a3, it = _sc_a(user, item, utab_t, item_table.T)
    ut_b3 = _sc_b(user, utab_t)
    ut_a = ut_a3.transpose(1, 0, 2).reshape(K, H)
    ut_b = ut_b3.transpose(1, 0, 2).reshape(K, H)
    out = _mm1(ut_a, it)
    return _mm2(ut_b, it, out)


# speedup vs baseline: 1.1246x; 1.1246x over previous
import functools

import jax
import jax.numpy as jnp
from jax import lax
from jax.experimental import pallas as pl
from jax.experimental.pallas import tpu as pltpu
from jax.experimental.pallas import tpu_sc as plsc

B = 4096
K = 32

_info = plsc.get_sparse_core_info()
_NC, _NS = _info.num_cores, _info.num_subcores
_NW = _NC * _NS
_BPW = B // _NW
_SG = 4
_NSG = _BPW // _SG


def _splat(x, n=16):
    return jnp.full((n,), x, jnp.int32)


def _sc_gather_body(user_idx, item_idx, utab_t, itab_t, out_ut, out_it,
                    idx_u, idx_i, buf_u, buf_i, cols_u, cols_i, sem_u, sem_i):
    wid = lax.axis_index("s") * _NC + lax.axis_index("c")
    base = wid * _BPW
    pltpu.sync_copy(user_idx.at[pl.ds(base, _BPW)], idx_u)
    pltpu.sync_copy(item_idx.at[pl.ds(base, _BPW)], idx_i)
    c_lo = lax.iota(jnp.int32, 16)
    c_hi = c_lo + 16

    vregs = []
    for h in range(_BPW // 16):
        vu = idx_u[pl.ds(h * 16, 16)]
        vi = idx_i[pl.ds(h * 16, 16)]
        vregs.append((
            lax.shift_right_logical(vu, 7), lax.bitwise_and(vu, _splat(127)),
            lax.shift_right_logical(vi, 7), lax.bitwise_and(vi, _splat(127)),
        ))

    def fire(g, q):
        tu, _, ti, _ = vregs[(g * _SG) // 16]
        for k in range(_SG):
            lane = (g * _SG) % 16 + k
            offu = pl.multiple_of(tu[lane] * 128, 128)
            offi = pl.multiple_of(ti[lane] * 128, 128)
            pltpu.make_async_copy(
                utab_t.at[:, pl.ds(offu, 128)], buf_u.at[q, k], sem_u).start()
            pltpu.make_async_copy(
                itab_t.at[:, pl.ds(offi, 128)], buf_i.at[q, k], sem_i).start()

    def drain_extract(g, q):
        _, ru, _, ri = vregs[(g * _SG) // 16]
        for k in range(_SG):
            pltpu.make_async_copy(
                utab_t.at[:, pl.ds(0, 128)], buf_u.at[q, k], sem_u).wait()
            pltpu.make_async_copy(
                itab_t.at[:, pl.ds(0, 128)], buf_i.at[q, k], sem_i).wait()
        qv = _splat(q)
        for k in range(_SG):
            lane = (g * _SG) % 16 + k
            slot = _splat(k)
            jv = _splat(g * _SG + k)
            rmu = _splat(ru[lane])
            rmi = _splat(ri[lane])
            u_lo = plsc.load_gather(buf_u, [qv, slot, c_lo, rmu])
            u_hi = plsc.load_gather(buf_u, [qv, slot, c_hi, rmu])
            i_lo = plsc.load_gather(buf_i, [qv, slot, c_lo, rmi])
            i_hi = plsc.load_gather(buf_i, [qv, slot, c_hi, rmi])
            plsc.store_scatter(cols_u, [c_lo, jv], u_lo)
            plsc.store_scatter(cols_u, [c_hi, jv], u_hi)
            plsc.store_scatter(cols_i, [c_lo, jv], i_lo)
            plsc.store_scatter(cols_i, [c_hi, jv], i_hi)

    fire(0, 0)
    for g in range(_NSG):
        if g + 1 < _NSG:
            fire(g + 1, (g + 1) % 2)
        drain_extract(g, g % 2)

    pltpu.sync_copy(cols_u, out_ut.at[:, pl.ds(base, _BPW)])
    pltpu.sync_copy(cols_i, out_it.at[:, pl.ds(base, _BPW)])


_sc_gather = functools.partial(
    pl.kernel,
    mesh=plsc.VectorSubcoreMesh(core_axis_name="c", subcore_axis_name="s"),
    out_type=(
        jax.ShapeDtypeStruct((K, B), jnp.float32),
        jax.ShapeDtypeStruct((K, B), jnp.float32),
    ),
    scratch_types=[
        pltpu.VMEM((_BPW,), jnp.int32),
        pltpu.VMEM((_BPW,), jnp.int32),
        pltpu.VMEM((2, _SG, K, 128), jnp.float32),
        pltpu.VMEM((2, _SG, K, 128), jnp.float32),
        pltpu.VMEM((K, _BPW), jnp.float32),
        pltpu.VMEM((K, _BPW), jnp.float32),
        pltpu.SemaphoreType.DMA,
        pltpu.SemaphoreType.DMA,
    ],
    compiler_params=pltpu.CompilerParams(needs_layout_passes=False),
)(_sc_gather_body)


_BM = 512
_BN = 4096


def _mm_body(u_ref, i_ref, o_ref):
    o_ref[...] = lax.dot_general(
        u_ref[...], i_ref[...],
        dimension_numbers=(((0,), (0,)), ((), ())),
        preferred_element_type=jnp.float32,
    )


_mm = pl.pallas_call(
    _mm_body,
    grid=(B // _BM, B // _BN),
    in_specs=[
        pl.BlockSpec((K, _BM), lambda i, j: (0, i)),
        pl.BlockSpec((K, _BN), lambda i, j: (0, j)),
    ],
    out_specs=pl.BlockSpec((_BM, _BN), lambda i, j: (i, j)),
    out_shape=jax.ShapeDtypeStruct((B, B), jnp.float32),
)


def kernel(user, item, user_table, item_table):
    ut, it = _sc_gather(user.astype(jnp.int32), item.astype(jnp.int32),
                        user_table.T, item_table.T)
    return _mm(ut, it)
